# Initial kernel scaffold; baseline (speedup 1.0000x reference)
import jax
import jax.numpy as jnp
from jax.experimental import pallas as pl


def kernel(x, edge_index, W_l1, b_l1, W_r1, W_l2, b_l2, W_r2, W_h1, b_h1, W_h2, b_h2):
    raise NotImplementedError("write your pallas kernel here")


# speedup vs baseline: 5.1915x; 5.1915x over previous
import functools

import jax
import jax.numpy as jnp
from jax import lax
from jax.experimental import pallas as pl
from jax.experimental.pallas import tpu as pltpu
from jax.experimental.pallas import tpu_sc as plsc

_N = 10000
_D = 128
_H = 64
_A = 4
_NPAD = 10240
_NC = 2
_NS = 16
_NW = _NC * _NS
_CHUNK = 128
_EPT_CHUNKS = 79
_EPT = _CHUNK * _EPT_CHUNKS
_EPAD = _EPT * _NW
_RPT = _NPAD // _NS
_BLK = 512
_GRID = _NPAD // _BLK


def _make_seg_sum(width):
  mesh = plsc.VectorSubcoreMesh(
      core_axis_name="c", subcore_axis_name="s",
      num_cores=_NC, num_subcores=_NS)

  @functools.partial(
      pl.kernel,
      out_type=jax.ShapeDtypeStruct((_NC * _NPAD, width), jnp.float32),
      mesh=mesh,
      compiler_params=pltpu.CompilerParams(use_tc_tiling_on_sc=False),
      scratch_types=[
          pltpu.VMEM((_CHUNK,), jnp.int32),
          pltpu.VMEM((_CHUNK,), jnp.int32),
          pltpu.VMEM((_CHUNK, width), jnp.float32),
          pltpu.VMEM((_CHUNK, width), jnp.float32),
          pltpu.VMEM_SHARED((_NPAD, width), jnp.float32),
          pltpu.SemaphoreType.DMA,
      ],
  )
  def seg_sum(p_hbm, src_hbm, dst_hbm, zeros_hbm, out_hbm,
              src_v, dst_v, rows_v, zbuf, acc_sh, sem):
    cid = lax.axis_index("c")
    sid = lax.axis_index("s")
    wid = cid * _NS + sid

    pltpu.sync_copy(zeros_hbm, zbuf)
    for r in range(_RPT // _CHUNK):
      pltpu.sync_copy(zbuf, acc_sh.at[pl.ds(sid * _RPT + r * _CHUNK, _CHUNK)])
    plsc.subcore_barrier()

    def step(i, carry):
      base = wid * _EPT + i * _CHUNK
      pltpu.sync_copy(src_hbm.at[pl.ds(base, _CHUNK)], src_v)
      pltpu.sync_copy(dst_hbm.at[pl.ds(base, _CHUNK)], dst_v)
      pltpu.async_copy(p_hbm.at[src_v], rows_v, sem).wait()
      pltpu.sync_copy(rows_v, acc_sh.at[dst_v], add=True)
      return carry

    lax.fori_loop(0, _EPT_CHUNKS, step, 0)
    plsc.subcore_barrier()

    pltpu.sync_copy(acc_sh.at[pl.ds(sid * _RPT, _RPT)],
                    out_hbm.at[pl.ds(cid * _NPAD + sid * _RPT, _RPT)])

  return seg_sum


_seg_sum80 = _make_seg_sum(80)
_seg_sum64 = _make_seg_sum(64)


def _tc_a_body(x_ref, wl_ref, wr_ref, ba_ref, p_ref, r_ref):
  xb = x_ref[...]
  p_ref[...] = jnp.dot(xb, wl_ref[...],
                       preferred_element_type=jnp.float32) + ba_ref[...]
  r_ref[...] = jnp.dot(xb, wr_ref[...], preferred_element_type=jnp.float32)


def _tc_b_body(a0_ref, a1_ref, r1_ref, b1_ref, wl_ref, wr_ref,
               p2_ref, r2_ref, ic_ref):
  a0 = a0_ref[...]
  a1 = a1_ref[...]
  s1 = a0[:, :_H] + a1[:, :_H]
  cnt = a0[:, _H:_H + 1] + a1[:, _H:_H + 1]
  ic = 1.0 / jnp.maximum(cnt, 1.0)
  h = jnp.maximum(s1 * ic + b1_ref[...] + r1_ref[...], 0.0)
  p2_ref[...] = jnp.dot(h, wl_ref[...], preferred_element_type=jnp.float32)
  r2_ref[...] = jnp.dot(h, wr_ref[...], preferred_element_type=jnp.float32)
  ic_ref[...] = jnp.broadcast_to(ic, (_BLK, _H))


def _tc_c_body(c0_ref, c1_ref, r2_ref, ic_ref, b2_ref, wh1_ref, bh1_ref,
               wh2_ref, bh2_ref, o_ref):
  s2 = c0_ref[...] + c1_ref[...]
  h2 = jnp.maximum(s2 * ic_ref[...] + b2_ref[...] + r2_ref[...], 0.0)
  h3 = jnp.maximum(
      jnp.dot(h2, wh1_ref[...], preferred_element_type=jnp.float32)
      + bh1_ref[...], 0.0)
  o_ref[...] = jnp.dot(h3, wh2_ref[...],
                       preferred_element_type=jnp.float32) + bh2_ref[...]


def _row_spec(width):
  return pl.BlockSpec((_BLK, width), lambda i: (i, 0))


def _full_spec(shape):
  return pl.BlockSpec(shape, lambda i: (0,) * len(shape))


def kernel(x, edge_index, W_l1, b_l1, W_r1, W_l2, b_l2, W_r2,
           W_h1, b_h1, W_h2, b_h2):
  f32 = jnp.float32
  x_pad = jnp.pad(x, ((0, _NPAD - _N), (0, 0)))
  src = edge_index[0].astype(jnp.int32)
  dst = edge_index[1].astype(jnp.int32)
  e = src.shape[0]
  src = jnp.concatenate([src, jnp.zeros((_EPAD - e,), jnp.int32)])
  dst = jnp.concatenate([dst, jnp.full((_EPAD - e,), _NPAD - 1, jnp.int32)])

  wl1 = jnp.pad(W_l1.T, ((0, 0), (0, 16)))
  ba = jnp.zeros((1, 80), f32).at[0, _H].set(1.0)
  zeros80 = jnp.zeros((_CHUNK, 80), f32)
  zeros64 = jnp.zeros((_CHUNK, _H), f32)

  p1, r1 = pl.pallas_call(
      _tc_a_body,
      grid=(_GRID,),
      in_specs=[_row_spec(_D), _full_spec((_D, 80)), _full_spec((_D, _H)),
                _full_spec((1, 80))],
      out_specs=[_row_spec(80), _row_spec(_H)],
      out_shape=[jax.ShapeDtypeStruct((_NPAD, 80), f32),
                 jax.ShapeDtypeStruct((_NPAD, _H), f32)],
  )(x_pad, wl1, W_r1.T, ba)

  acc1 = _seg_sum80(p1, src, dst, zeros80)
  a0, a1 = acc1[:_NPAD], acc1[_NPAD:]

  p2, r2, ic = pl.pallas_call(
      _tc_b_body,
      grid=(_GRID,),
      in_specs=[_row_spec(80), _row_spec(80), _row_spec(_H),
                _full_spec((1, _H)), _full_spec((_H, _H)),
                _full_spec((_H, _H))],
      out_specs=[_row_spec(_H), _row_spec(_H), _row_spec(_H)],
      out_shape=[jax.ShapeDtypeStruct((_NPAD, _H), f32)] * 3,
  )(a0, a1, r1, b_l1.reshape(1, _H), W_l2.T, W_r2.T)

  acc2 = _seg_sum64(p2, src, dst, zeros64)
  c0, c1 = acc2[:_NPAD], acc2[_NPAD:]

  wh2 = jnp.pad(W_h2.T, ((0, 0), (0, 128 - _A)))
  bh2 = jnp.pad(b_h2.reshape(1, _A), ((0, 0), (0, 128 - _A)))

  outp = pl.pallas_call(
      _tc_c_body,
      grid=(_GRID,),
      in_specs=[_row_spec(_H), _row_spec(_H), _row_spec(_H), _row_spec(_H),
                _full_spec((1, _H)), _full_spec((_H, _H)),
                _full_spec((1, _H)), _full_spec((_H, 128)),
                _full_spec((1, 128))],
      out_specs=_row_spec(128),
      out_shape=jax.ShapeDtypeStruct((_NPAD, 128), f32),
  )(c0, c1, r2, ic, b_l2.reshape(1, _H), W_h1.T, b_h1.reshape(1, _H),
    wh2, bh2)

  return outp[:_N, :_A]


# speedup vs baseline: 5.4915x; 1.0578x over previous
import functools

import jax
import jax.numpy as jnp
from jax import lax
from jax.experimental import pallas as pl
from jax.experimental.pallas import tpu as pltpu
from jax.experimental.pallas import tpu_sc as plsc

_N = 10000
_D = 128
_H = 64
_A = 4
_NPAD = 10240
_NC = 2
_NS = 16
_NW = _NC * _NS
_CHUNK = 128
_EPT_CHUNKS = 80
_EPT = _CHUNK * _EPT_CHUNKS
_EPAD = _EPT * _NW
_NB = 4
_RPT = _NPAD // _NS
_BLK = 512
_GRID = _NPAD // _BLK


def _make_seg_sum(width):
  mesh = plsc.VectorSubcoreMesh(
      core_axis_name="c", subcore_axis_name="s",
      num_cores=_NC, num_subcores=_NS)

  @functools.partial(
      pl.kernel,
      out_type=jax.ShapeDtypeStruct((_NC * _NPAD, width), jnp.float32),
      mesh=mesh,
      compiler_params=pltpu.CompilerParams(use_tc_tiling_on_sc=False),
      scratch_types=[
          pltpu.VMEM((_EPT_CHUNKS, _CHUNK), jnp.int32),
          pltpu.VMEM((_EPT_CHUNKS, _CHUNK), jnp.int32),
          pltpu.VMEM((_NB, _CHUNK, width), jnp.float32),
          pltpu.VMEM((_CHUNK, width), jnp.float32),
          pltpu.VMEM_SHARED((_NPAD, width), jnp.float32),
          [pltpu.SemaphoreType.DMA] * _NB,
      ],
  )
  def seg_sum(p_hbm, src_hbm, dst_hbm, zeros_hbm, out_hbm,
              src_v, dst_v, rows_v, zbuf, acc_sh, sems):
    cid = lax.axis_index("c")
    sid = lax.axis_index("s")
    wid = cid * _NS + sid

    pltpu.sync_copy(src_hbm.at[pl.ds(wid * _EPT_CHUNKS, _EPT_CHUNKS)], src_v)
    pltpu.sync_copy(dst_hbm.at[pl.ds(wid * _EPT_CHUNKS, _EPT_CHUNKS)], dst_v)

    pltpu.sync_copy(zeros_hbm, zbuf)
    for r in range(_RPT // _CHUNK):
      pltpu.sync_copy(zbuf, acc_sh.at[pl.ds(sid * _RPT + r * _CHUNK, _CHUNK)])
    plsc.subcore_barrier()

    for b in range(_NB):
      pltpu.async_copy(p_hbm.at[src_v.at[b]], rows_v.at[b], sems[b])

    def outer(i, carry):
      for b in range(_NB):
        j = i * _NB + b
        pltpu.make_async_copy(p_hbm.at[src_v.at[j]], rows_v.at[b],
                              sems[b]).wait()
        pltpu.sync_copy(rows_v.at[b], acc_sh.at[dst_v.at[j]], add=True)

        @pl.when(j + _NB < _EPT_CHUNKS)
        def _():
          pltpu.async_copy(p_hbm.at[src_v.at[j + _NB]], rows_v.at[b], sems[b])
      return carry

    lax.fori_loop(0, _EPT_CHUNKS // _NB, outer, 0)
    plsc.subcore_barrier()

    pltpu.sync_copy(acc_sh.at[pl.ds(sid * _RPT, _RPT)],
                    out_hbm.at[pl.ds(cid * _NPAD + sid * _RPT, _RPT)])

  return seg_sum


_seg_sum80 = _make_seg_sum(80)
_seg_sum64 = _make_seg_sum(64)


def _tc_a_body(x_ref, wl_ref, wr_ref, ba_ref, p_ref, r_ref):
  xb = x_ref[...]
  p_ref[...] = jnp.dot(xb, wl_ref[...],
                       preferred_element_type=jnp.float32) + ba_ref[...]
  r_ref[...] = jnp.dot(xb, wr_ref[...], preferred_element_type=jnp.float32)


def _tc_b_body(a0_ref, a1_ref, r1_ref, b1_ref, wl_ref, wr_ref,
               p2_ref, r2_ref, ic_ref):
  a0 = a0_ref[...]
  a1 = a1_ref[...]
  s1 = a0[:, :_H] + a1[:, :_H]
  cnt = a0[:, _H:_H + 1] + a1[:, _H:_H + 1]
  ic = 1.0 / jnp.maximum(cnt, 1.0)
  h = jnp.maximum(s1 * ic + b1_ref[...] + r1_ref[...], 0.0)
  p2_ref[...] = jnp.dot(h, wl_ref[...], preferred_element_type=jnp.float32)
  r2_ref[...] = jnp.dot(h, wr_ref[...], preferred_element_type=jnp.float32)
  ic_ref[...] = jnp.broadcast_to(ic, (_BLK, _H))


def _tc_c_body(c0_ref, c1_ref, r2_ref, ic_ref, b2_ref, wh1_ref, bh1_ref,
               wh2_ref, bh2_ref, o_ref):
  s2 = c0_ref[...] + c1_ref[...]
  h2 = jnp.maximum(s2 * ic_ref[...] + b2_ref[...] + r2_ref[...], 0.0)
  h3 = jnp.maximum(
      jnp.dot(h2, wh1_ref[...], preferred_element_type=jnp.float32)
      + bh1_ref[...], 0.0)
  o_ref[...] = jnp.dot(h3, wh2_ref[...],
                       preferred_element_type=jnp.float32) + bh2_ref[...]


def _row_spec(width):
  return pl.BlockSpec((_BLK, width), lambda i: (i, 0))


def _full_spec(shape):
  return pl.BlockSpec(shape, lambda i: (0,) * len(shape))


def kernel(x, edge_index, W_l1, b_l1, W_r1, W_l2, b_l2, W_r2,
           W_h1, b_h1, W_h2, b_h2):
  f32 = jnp.float32
  x_pad = jnp.pad(x, ((0, _NPAD - _N), (0, 0)))
  src = edge_index[0].astype(jnp.int32)
  dst = edge_index[1].astype(jnp.int32)
  e = src.shape[0]
  src = jnp.concatenate(
      [src, jnp.zeros((_EPAD - e,), jnp.int32)]).reshape(-1, _CHUNK)
  dst = jnp.concatenate(
      [dst, jnp.full((_EPAD - e,), _NPAD - 1, jnp.int32)]).reshape(-1, _CHUNK)

  wl1 = jnp.pad(W_l1.T, ((0, 0), (0, 16)))
  ba = jnp.zeros((1, 80), f32).at[0, _H].set(1.0)
  zeros80 = jnp.zeros((_CHUNK, 80), f32)
  zeros64 = jnp.zeros((_CHUNK, _H), f32)

  p1, r1 = pl.pallas_call(
      _tc_a_body,
      grid=(_GRID,),
      in_specs=[_row_spec(_D), _full_spec((_D, 80)), _full_spec((_D, _H)),
                _full_spec((1, 80))],
      out_specs=[_row_spec(80), _row_spec(_H)],
      out_shape=[jax.ShapeDtypeStruct((_NPAD, 80), f32),
                 jax.ShapeDtypeStruct((_NPAD, _H), f32)],
  )(x_pad, wl1, W_r1.T, ba)

  acc1 = _seg_sum80(p1, src, dst, zeros80)
  a0, a1 = acc1[:_NPAD], acc1[_NPAD:]

  p2, r2, ic = pl.pallas_call(
      _tc_b_body,
      grid=(_GRID,),
      in_specs=[_row_spec(80), _row_spec(80), _row_spec(_H),
                _full_spec((1, _H)), _full_spec((_H, _H)),
                _full_spec((_H, _H))],
      out_specs=[_row_spec(_H), _row_spec(_H), _row_spec(_H)],
      out_shape=[jax.ShapeDtypeStruct((_NPAD, _H), f32)] * 3,
  )(a0, a1, r1, b_l1.reshape(1, _H), W_l2.T, W_r2.T)

  acc2 = _seg_sum64(p2, src, dst, zeros64)
  c0, c1 = acc2[:_NPAD], acc2[_NPAD:]

  wh2 = jnp.pad(W_h2.T, ((0, 0), (0, 128 - _A)))
  bh2 = jnp.pad(b_h2.reshape(1, _A), ((0, 0), (0, 128 - _A)))

  outp = pl.pallas_call(
      _tc_c_body,
      grid=(_GRID,),
      in_specs=[_row_spec(_H), _row_spec(_H), _row_spec(_H), _row_spec(_H),
                _full_spec((1, _H)), _full_spec((_H, _H)),
                _full_spec((1, _H)), _full_spec((_H, 128)),
                _full_spec((1, 128))],
      out_specs=_row_spec(128),
      out_shape=jax.ShapeDtypeStruct((_NPAD, 128), f32),
  )(c0, c1, r2, ic, b_l2.reshape(1, _H), W_h1.T, b_h1.reshape(1, _H),
    wh2, bh2)

  return outp[:_N, :_A]
